# baseline (speedup 1.0000x reference)
import functools

import jax
import jax.numpy as jnp
from jax import lax
from jax.experimental import pallas as pl
from jax.experimental.pallas import tpu as pltpu
from jax.experimental.pallas import tpu_sc as plsc

EMBED_DIM = 64
NUM_CORES = 2
NUM_SUBCORES = 16
NUM_WORKERS = NUM_CORES * NUM_SUBCORES
CHUNK = 128
NBUF = 8


def _table_transpose(table_t):
  v = table_t.shape[1]
  blk = 16384

  def body(in_ref, out_ref):
    eye = jnp.eye(EMBED_DIM, dtype=jnp.float32)
    out_ref[...] = jax.lax.dot_general(
        in_ref[...], eye, dimension_numbers=(((0,), (0,)), ((), ())),
        preferred_element_type=jnp.float32)

  return pl.pallas_call(
      body,
      grid=(pl.cdiv(v, blk),),
      in_specs=[pl.BlockSpec((EMBED_DIM, blk), lambda i: (0, i))],
      out_specs=pl.BlockSpec((blk, EMBED_DIM), lambda i: (i, 0)),
      out_shape=jax.ShapeDtypeStruct((v, EMBED_DIM), jnp.float32),
  )(table_t)


def _idx_flatten(idx_t):
  fields, batch = idx_t.shape

  def body(in_ref, out_ref):
    for f in range(fields):
      out_ref[pl.ds(f * batch, batch)] = in_ref[f]

  return pl.pallas_call(
      body,
      in_specs=[pl.BlockSpec((fields, batch), lambda: (0, 0))],
      out_specs=pl.BlockSpec((fields * batch,), lambda: (0,)),
      out_shape=jax.ShapeDtypeStruct((fields * batch,), jnp.int32),
  )(idx_t)


def _out_transpose(rows_fb, fields, batch):
  blk = 8192
  assert batch % blk == 0
  per_f = batch // blk

  def body(in_ref, out_ref):
    eye = jnp.eye(EMBED_DIM, dtype=jnp.float32)
    out_ref[0] = jax.lax.dot_general(
        eye, in_ref[...], dimension_numbers=(((0,), (1,)), ((), ())),
        preferred_element_type=jnp.float32)

  return pl.pallas_call(
      body,
      grid=(fields, per_f),
      in_specs=[
          pl.BlockSpec((blk, EMBED_DIM), lambda f, i: (f * per_f + i, 0))
      ],
      out_specs=pl.BlockSpec((1, EMBED_DIM, blk), lambda f, i: (f, 0, i)),
      out_shape=jax.ShapeDtypeStruct((fields, EMBED_DIM, batch), jnp.float32),
  )(rows_fb)


@functools.lru_cache(maxsize=None)
def _build_gather(batch_total: int):
  assert batch_total % (NUM_WORKERS * NBUF * CHUNK) == 0
  b_per_w = batch_total // NUM_WORKERS
  n_rounds = b_per_w // (NBUF * CHUNK)
  mesh = plsc.VectorSubcoreMesh(core_axis_name="c", subcore_axis_name="s")

  scratch = (
      [pltpu.VMEM((CHUNK,), jnp.int32) for _ in range(NBUF)]
      + [pltpu.VMEM((CHUNK, EMBED_DIM), jnp.float32) for _ in range(NBUF)]
      + [pltpu.SemaphoreType.DMA for _ in range(3 * NBUF)]
  )

  @functools.partial(
      pl.kernel,
      mesh=mesh,
      out_type=jax.ShapeDtypeStruct((batch_total, EMBED_DIM), jnp.float32),
      scratch_types=scratch,
      compiler_params=pltpu.CompilerParams(use_tc_tiling_on_sc=False),
  )
  def gather_kernel(table_hbm, idx_hbm, out_hbm, *scr):
    stage = scr[:NBUF]
    rows = scr[NBUF:2 * NBUF]
    isem = scr[2 * NBUF:3 * NBUF]
    gsem = scr[3 * NBUF:4 * NBUF]
    wsem = scr[4 * NBUF:5 * NBUF]
    wid = lax.axis_index("s") * NUM_CORES + lax.axis_index("c")
    base = wid * b_per_w

    def idx_copy(i, s):
      return pltpu.make_async_copy(idx_hbm.at[pl.ds(base + i * CHUNK, CHUNK)],
                                   stage[s], isem[s])

    def gather(i, s):
      del i
      return pltpu.make_async_copy(table_hbm.at[stage[s]], rows[s], gsem[s])

    def write(i, s):
      return pltpu.make_async_copy(
          rows[s], out_hbm.at[pl.ds(base + i * CHUNK, CHUNK)], wsem[s])

    for s in range(NBUF):
      idx_copy(s, s).start()

    def body(r, _):
      i0 = r * NBUF
      for s in range(NBUF):
        idx_copy(i0 + s, s).wait()

        @pl.when(r > 0)
        def _():
          write(i0 + s - NBUF, s).wait()

        gather(i0 + s, s).start()
      for s in range(NBUF):
        gather(i0 + s, s).wait()
        write(i0 + s, s).start()

        @pl.when(r + 1 < n_rounds)
        def _():
          idx_copy(i0 + s + NBUF, s).start()

      return 0

    lax.fori_loop(0, n_rounds, body, 0)
    for s in range(NBUF):
      write((n_rounds - 1) * NBUF + s, s).wait()

  return gather_kernel


def kernel(inputs, embeddings):
  batch, fields = inputs.shape
  idx_flat = _idx_flatten(inputs.T.astype(jnp.int32))
  rows_fb = _build_gather(fields * batch)(embeddings, idx_flat)
  out_t = _out_transpose(rows_fb, fields, batch)
  return jnp.transpose(out_t, (2, 0, 1))


# speedup vs baseline: 1.6007x; 1.6007x over previous
import functools

import jax
import jax.numpy as jnp
from jax import lax
from jax.experimental import pallas as pl
from jax.experimental.pallas import tpu as pltpu
from jax.experimental.pallas import tpu_sc as plsc

EMBED_DIM = 64
NUM_CORES = 2
NUM_SUBCORES = 16
NUM_WORKERS = NUM_CORES * NUM_SUBCORES
CHUNK = 128
NBUF = 4
ROW_W = 2 * EMBED_DIM


def _table_transpose(table_t):
  v = table_t.shape[1]
  blk = 16384

  def body(in_ref, out_ref):
    eye = jnp.eye(EMBED_DIM, dtype=jnp.float32)
    t = jax.lax.dot_general(
        in_ref[...], eye, dimension_numbers=(((0,), (0,)), ((), ())),
        preferred_element_type=jnp.float32)
    out_ref[:, 0:EMBED_DIM] = t
    out_ref[:, EMBED_DIM:2 * EMBED_DIM] = t

  return pl.pallas_call(
      body,
      grid=(pl.cdiv(v, blk),),
      in_specs=[pl.BlockSpec((EMBED_DIM, blk), lambda i: (0, i))],
      out_specs=pl.BlockSpec((blk, 2 * EMBED_DIM), lambda i: (i, 0)),
      out_shape=jax.ShapeDtypeStruct((v, 2 * EMBED_DIM), jnp.float32),
  )(table_t)


def _idx_flatten(idx_t):
  fields, batch = idx_t.shape

  def body(in_ref, out_ref):
    for f in range(fields):
      out_ref[pl.ds(f * batch, batch)] = in_ref[f]

  return pl.pallas_call(
      body,
      in_specs=[pl.BlockSpec((fields, batch), lambda: (0, 0))],
      out_specs=pl.BlockSpec((fields * batch,), lambda: (0,)),
      out_shape=jax.ShapeDtypeStruct((fields * batch,), jnp.int32),
  )(idx_t)


def _out_transpose(rows_fb2, fields, batch):
  blk = 4096
  assert batch % blk == 0
  per_f = batch // blk

  def body(in_ref, out_ref):
    eye = jnp.eye(EMBED_DIM, dtype=jnp.float32)
    x = in_ref[:, 0:EMBED_DIM]
    out_ref[0] = jax.lax.dot_general(
        eye, x, dimension_numbers=(((0,), (1,)), ((), ())),
        preferred_element_type=jnp.float32)

  return pl.pallas_call(
      body,
      grid=(fields, per_f),
      in_specs=[
          pl.BlockSpec((blk, 2 * EMBED_DIM), lambda f, i: (f * per_f + i, 0))
      ],
      out_specs=pl.BlockSpec((1, EMBED_DIM, blk), lambda f, i: (f, 0, i)),
      out_shape=jax.ShapeDtypeStruct((fields, EMBED_DIM, batch), jnp.float32),
  )(rows_fb2)


@functools.lru_cache(maxsize=None)
def _build_gather(batch_total: int):
  assert batch_total % (NUM_WORKERS * NBUF * CHUNK) == 0
  b_per_w = batch_total // NUM_WORKERS
  n_rounds = b_per_w // (NBUF * CHUNK)
  mesh = plsc.VectorSubcoreMesh(core_axis_name="c", subcore_axis_name="s")

  scratch = (
      [pltpu.VMEM((CHUNK,), jnp.int32) for _ in range(NBUF)]
      + [pltpu.VMEM((CHUNK, ROW_W), jnp.float32) for _ in range(NBUF)]
      + [pltpu.SemaphoreType.DMA for _ in range(3 * NBUF)]
  )

  @functools.partial(
      pl.kernel,
      mesh=mesh,
      out_type=jax.ShapeDtypeStruct((batch_total, ROW_W), jnp.float32),
      scratch_types=scratch,
      compiler_params=pltpu.CompilerParams(use_tc_tiling_on_sc=False),
  )
  def gather_kernel(table_hbm, idx_hbm, out_hbm, *scr):
    stage = scr[:NBUF]
    rows = scr[NBUF:2 * NBUF]
    isem = scr[2 * NBUF:3 * NBUF]
    gsem = scr[3 * NBUF:4 * NBUF]
    wsem = scr[4 * NBUF:5 * NBUF]
    wid = lax.axis_index("s") * NUM_CORES + lax.axis_index("c")
    base = wid * b_per_w

    def idx_copy(i, s):
      return pltpu.make_async_copy(idx_hbm.at[pl.ds(base + i * CHUNK, CHUNK)],
                                   stage[s], isem[s])

    def gather(i, s):
      del i
      return pltpu.make_async_copy(table_hbm.at[stage[s]], rows[s], gsem[s])

    def write(i, s):
      return pltpu.make_async_copy(
          rows[s], out_hbm.at[pl.ds(base + i * CHUNK, CHUNK)], wsem[s])

    for s in range(NBUF):
      idx_copy(s, s).start()

    def body(r, _):
      i0 = r * NBUF
      for s in range(NBUF):
        idx_copy(i0 + s, s).wait()

        @pl.when(r > 0)
        def _():
          write(i0 + s - NBUF, s).wait()

        gather(i0 + s, s).start()
      for s in range(NBUF):
        gather(i0 + s, s).wait()
        write(i0 + s, s).start()

        @pl.when(r + 1 < n_rounds)
        def _():
          idx_copy(i0 + s + NBUF, s).start()

      return 0

    lax.fori_loop(0, n_rounds, body, 0)
    for s in range(NBUF):
      write((n_rounds - 1) * NBUF + s, s).wait()

  return gather_kernel


def kernel(inputs, embeddings):
  batch, fields = inputs.shape
  table_pad = _table_transpose(embeddings.T)
  idx_flat = _idx_flatten(inputs.T.astype(jnp.int32))
  rows_fb = _build_gather(fields * batch)(table_pad, idx_flat)
  out_t = _out_transpose(rows_fb, fields, batch)
  return jnp.transpose(out_t, (2, 0, 1))


# speedup vs baseline: 1.6138x; 1.0082x over previous
import functools

import jax
import jax.numpy as jnp
from jax import lax
from jax.experimental import pallas as pl
from jax.experimental.pallas import tpu as pltpu
from jax.experimental.pallas import tpu_sc as plsc

EMBED_DIM = 64
NUM_CORES = 2
NUM_SUBCORES = 16
NUM_WORKERS = NUM_CORES * NUM_SUBCORES
CHUNK = 128
NBUF = 4
ROW_W = 2 * EMBED_DIM


def _table_transpose(table_t):
  v = table_t.shape[1]
  blk = 16384

  def body(in_ref, out_ref):
    eye = jnp.eye(EMBED_DIM, dtype=jnp.float32)
    t = jax.lax.dot_general(
        in_ref[...], eye, dimension_numbers=(((0,), (0,)), ((), ())),
        preferred_element_type=jnp.float32)
    out_ref[:, 0:EMBED_DIM] = t
    out_ref[:, EMBED_DIM:2 * EMBED_DIM] = t

  return pl.pallas_call(
      body,
      grid=(pl.cdiv(v, blk),),
      in_specs=[pl.BlockSpec((EMBED_DIM, blk), lambda i: (0, i))],
      out_specs=pl.BlockSpec((blk, 2 * EMBED_DIM), lambda i: (i, 0)),
      out_shape=jax.ShapeDtypeStruct((v, 2 * EMBED_DIM), jnp.float32),
  )(table_t)


def _idx_flatten(idx_t):
  fields, batch = idx_t.shape

  def body(in_ref, out_ref):
    for f in range(fields):
      out_ref[pl.ds(f * batch, batch)] = in_ref[f]

  return pl.pallas_call(
      body,
      in_specs=[pl.BlockSpec((fields, batch), lambda: (0, 0))],
      out_specs=pl.BlockSpec((fields * batch,), lambda: (0,)),
      out_shape=jax.ShapeDtypeStruct((fields * batch,), jnp.int32),
  )(idx_t)


def _out_transpose(rows_fb2, fields, batch):
  blk = 4096
  assert batch % blk == 0
  per_f = batch // blk

  def body(in_ref, out_ref):
    eye = jnp.eye(EMBED_DIM, dtype=jnp.float32)
    x = in_ref[:, 0:EMBED_DIM]
    out_ref[0] = jax.lax.dot_general(
        eye, x, dimension_numbers=(((0,), (1,)), ((), ())),
        preferred_element_type=jnp.float32)

  return pl.pallas_call(
      body,
      grid=(fields, per_f),
      in_specs=[
          pl.BlockSpec((blk, 2 * EMBED_DIM), lambda f, i: (f * per_f + i, 0))
      ],
      out_specs=pl.BlockSpec((1, EMBED_DIM, blk), lambda f, i: (f, 0, i)),
      out_shape=jax.ShapeDtypeStruct((fields, EMBED_DIM, batch), jnp.float32),
  )(rows_fb2)


@functools.lru_cache(maxsize=None)
def _build_gather(batch_total: int):
  assert batch_total % (NUM_WORKERS * NBUF * CHUNK) == 0
  b_per_w = batch_total // NUM_WORKERS
  n_rounds = b_per_w // (NBUF * CHUNK)
  mesh = plsc.VectorSubcoreMesh(core_axis_name="c", subcore_axis_name="s")

  scratch = (
      [pltpu.VMEM((CHUNK,), jnp.int32) for _ in range(NBUF)]
      + [pltpu.VMEM((CHUNK, ROW_W), jnp.float32) for _ in range(NBUF)]
      + [pltpu.SemaphoreType.DMA for _ in range(3 * NBUF)]
  )

  @functools.partial(
      pl.kernel,
      mesh=mesh,
      out_type=jax.ShapeDtypeStruct((batch_total, ROW_W), jnp.float32),
      scratch_types=scratch,
      compiler_params=pltpu.CompilerParams(use_tc_tiling_on_sc=False),
  )
  def gather_kernel(table_hbm, idx_hbm, out_hbm, *scr):
    stage = scr[:NBUF]
    rows = scr[NBUF:2 * NBUF]
    isem = scr[2 * NBUF:3 * NBUF]
    gsem = scr[3 * NBUF:4 * NBUF]
    wsem = scr[4 * NBUF:5 * NBUF]
    wid = lax.axis_index("s") * NUM_CORES + lax.axis_index("c")
    base = wid * b_per_w

    def idx_copy(i, s):
      return pltpu.make_async_copy(idx_hbm.at[pl.ds(base + i * CHUNK, CHUNK)],
                                   stage[s], isem[s])

    def gather(i, s):
      del i
      return pltpu.make_async_copy(table_hbm.at[stage[s]], rows[s], gsem[s])

    def write(i, s):
      return pltpu.make_async_copy(
          rows[s].at[:, pl.ds(0, EMBED_DIM)],
          out_hbm.at[pl.ds(base + i * CHUNK, CHUNK), pl.ds(0, EMBED_DIM)],
          wsem[s])

    for s in range(NBUF):
      idx_copy(s, s).start()

    def body(r, _):
      i0 = r * NBUF
      for s in range(NBUF):
        idx_copy(i0 + s, s).wait()

        @pl.when(r > 0)
        def _():
          write(i0 + s - NBUF, s).wait()

        gather(i0 + s, s).start()
      for s in range(NBUF):
        gather(i0 + s, s).wait()
        write(i0 + s, s).start()

        @pl.when(r + 1 < n_rounds)
        def _():
          idx_copy(i0 + s + NBUF, s).start()

      return 0

    lax.fori_loop(0, n_rounds, body, 0)
    for s in range(NBUF):
      write((n_rounds - 1) * NBUF + s, s).wait()

  return gather_kernel


def kernel(inputs, embeddings):
  batch, fields = inputs.shape
  table_pad = _table_transpose(embeddings.T)
  idx_flat = _idx_flatten(inputs.T.astype(jnp.int32))
  rows_fb = _build_gather(fields * batch)(table_pad, idx_flat)
  out_t = _out_transpose(rows_fb, fields, batch)
  return jnp.transpose(out_t, (2, 0, 1))
